# Initial kernel scaffold; baseline (speedup 1.0000x reference)
import jax
import jax.numpy as jnp
from jax.experimental import pallas as pl


def kernel(feat_p2, feat_p3, feat_p4, feat_p5, boxes0, boxes1):
    raise NotImplementedError("write your pallas kernel here")


# speedup vs baseline: 44.0196x; 44.0196x over previous
import functools

import numpy as np
import jax
import jax.numpy as jnp
from jax import lax
from jax.experimental import pallas as pl
from jax.experimental.pallas import tpu as pltpu
from jax.experimental.pallas import tpu_sc as plsc

_POOL = 7
_R = 1024
_C = 256
_NP = 784
_CHUNK = 112
_NCHUNK = 7
_NW = 32
_RPW = _R // _NW

def _col_patterns():
    p = lax.broadcasted_iota(jnp.int32, (1, _NP), 1)
    cxc = p % 2
    cyc = (p // 2) % 2
    sx = (p // 4) % 2
    sy = (p // 8) % 2
    pw = (p // 16) % 7
    ph = p // 112
    cxt = pw.astype(jnp.float32) + sx.astype(jnp.float32) * 0.5 + 0.25
    cyt = ph.astype(jnp.float32) + sy.astype(jnp.float32) * 0.5 + 0.25
    return cxt, cyt, cxc, cyc


def _p1_body(boxes_ref, idx_ref, w_ref):
    pid = pl.program_id(0)
    _CXT, _CYT, _CXC, _CYC = _col_patterns()
    x1 = boxes_ref[:, 0:1]
    y1 = boxes_ref[:, 1:2]
    x2 = boxes_ref[:, 2:3]
    y2 = boxes_ref[:, 3:4]
    area = (y2 - y1) * (x2 - x1)
    lvl = ((area >= 12544.0).astype(jnp.int32)
           + (area >= 50176.0).astype(jnp.int32)
           + (area >= 200704.0).astype(jnp.int32))
    scale = jnp.where(lvl == 0, 0.25,
                      jnp.where(lvl == 1, 0.125,
                                jnp.where(lvl == 2, 0.0625, 0.03125)))
    hw = jnp.right_shift(jnp.int32(256), lvl)
    hwf = hw.astype(jnp.float32)
    base = (jnp.where(lvl >= 1, 131072, 0)
            + jnp.where(lvl >= 2, 32768, 0)
            + jnp.where(lvl >= 3, 8192, 0))
    b = jnp.where(pid >= 4, 1, 0)

    def axis(lo, hi, ct, corner):
        los = lo * scale
        his = hi * scale
        ext = jnp.maximum(his - los, 1.0)
        bw = ext / float(_POOL)
        t = los + jnp.asarray(ct) * bw
        validf = ((t > -1.0) & (t < hwf)).astype(jnp.float32)
        tc = jnp.clip(t, 0.0, hwf - 1.0)
        i0 = jnp.floor(tc).astype(jnp.int32)
        i1 = jnp.minimum(i0 + 1, hw - 1)
        frac = tc - i0.astype(jnp.float32)
        cmask = jnp.asarray(corner) == 1
        sel = jnp.where(cmask, i1, i0)
        wgt = jnp.where(cmask, frac, 1.0 - frac) * validf
        return sel, wgt

    xsel, wx = axis(x1, x2, _CXT, _CXC)
    ysel, wy = axis(y1, y2, _CYT, _CYC)
    idx_ref[...] = base + (b * hw + ysel) * hw + xsel
    w_ref[...] = wx * wy * 0.25


def _phase1(boxes):
    return pl.pallas_call(
        _p1_body,
        grid=(8,),
        in_specs=[pl.BlockSpec((128, 4), lambda i: (i, 0))],
        out_specs=[pl.BlockSpec((128, _NP), lambda i: (i, 0)),
                   pl.BlockSpec((128, _NP), lambda i: (i, 0))],
        out_shape=[jax.ShapeDtypeStruct((_R, _NP), jnp.int32),
                   jax.ShapeDtypeStruct((_R, _NP), jnp.float32)],
    )(boxes)


def _lane_bcast(v, j):
    idxs = jnp.full((16, 1), j, jnp.int32)
    return lax.gather(
        v, idxs,
        lax.GatherDimensionNumbers(offset_dims=(), collapsed_slice_dims=(0,),
                                   start_index_map=(0,)),
        (1,), mode=lax.GatherScatterMode.PROMISE_IN_BOUNDS)


def _sc_body(idx_hbm, w_hbm, table_hbm, out_hbm, idx_v, w_v, rows_v, stage_v, sem):
    nc = 2
    wid = lax.axis_index("s") * nc + lax.axis_index("c")

    def roi_body(r_local, _):
        r = wid * _RPW + r_local
        pltpu.sync_copy(idx_hbm.at[pl.ds(r * _NP, _NP)], idx_v)
        pltpu.sync_copy(w_hbm.at[pl.ds(r * _NP, _NP)], w_v)

        def chunk_body(c, _):
            pltpu.async_copy(
                table_hbm.at[idx_v.at[pl.ds(c * _CHUNK, _CHUNK)]], rows_v, sem
            ).wait()

            def bin_body(bl, _):
                wvec = w_v[pl.ds(c * _CHUNK + bl * 16, 16)]
                acc = [jnp.zeros((16,), jnp.float32) for _ in range(16)]
                for j in range(16):
                    wj = _lane_bcast(wvec, j)
                    for k in range(16):
                        acc[k] = acc[k] + wj * rows_v[bl * 16 + j, pl.ds(k * 16, 16)]
                for k in range(16):
                    stage_v[c * 7 + bl, pl.ds(k * 16, 16)] = acc[k]
                return ()

            lax.fori_loop(0, 7, bin_body, (), unroll=False)
            return ()

        lax.fori_loop(0, _NCHUNK, chunk_body, (), unroll=False)
        pltpu.sync_copy(stage_v, out_hbm.at[pl.ds(r * 56, 56)])
        return ()

    lax.fori_loop(0, _RPW, roi_body, (), unroll=False)


def _phase2(idx, w, table):
    mesh = plsc.VectorSubcoreMesh(core_axis_name="c", subcore_axis_name="s")
    f = functools.partial(
        pl.kernel, _sc_body, mesh=mesh,
        out_type=jax.ShapeDtypeStruct((_R * 56, _C), jnp.float32),
        scratch_types=[
            pltpu.VMEM((_NP,), jnp.int32),
            pltpu.VMEM((_NP,), jnp.float32),
            pltpu.VMEM((_CHUNK, _C), jnp.float32),
            pltpu.VMEM((56, _C), jnp.float32),
            pltpu.SemaphoreType.DMA,
        ])()
    return f(idx, w, table)


def kernel(feat_p2, feat_p3, feat_p4, feat_p5, boxes0, boxes1):
    feats = [feat_p2, feat_p3, feat_p4, feat_p5]
    table = jnp.concatenate(
        [jnp.transpose(f, (0, 2, 3, 1)).reshape(-1, _C) for f in feats], axis=0)
    boxes = jnp.concatenate([boxes0, boxes1], axis=0)
    idx, w = _phase1(boxes)
    out_rows = _phase2(idx.reshape(-1), w.reshape(-1), table)
    out_rows = out_rows.reshape(_R, 56, _C)[:, :49, :]
    return jnp.transpose(out_rows.reshape(_R, _POOL, _POOL, _C), (0, 3, 1, 2))


# speedup vs baseline: 68.3387x; 1.5525x over previous
import functools

import numpy as np
import jax
import jax.numpy as jnp
from jax import lax
from jax.experimental import pallas as pl
from jax.experimental.pallas import tpu as pltpu
from jax.experimental.pallas import tpu_sc as plsc

_POOL = 7
_R = 1024
_C = 256
_NP = 784
_CHUNK = 112
_NCHUNK = 7
_NW = 32
_RPW = _R // _NW

def _col_patterns():
    p = lax.broadcasted_iota(jnp.int32, (1, _NP), 1)
    cxc = p % 2
    cyc = (p // 2) % 2
    sx = (p // 4) % 2
    sy = (p // 8) % 2
    pw = (p // 16) % 7
    ph = p // 112
    cxt = pw.astype(jnp.float32) + sx.astype(jnp.float32) * 0.5 + 0.25
    cyt = ph.astype(jnp.float32) + sy.astype(jnp.float32) * 0.5 + 0.25
    return cxt, cyt, cxc, cyc


def _p1_body(boxes_ref, idx_ref, w_ref):
    pid = pl.program_id(0)
    _CXT, _CYT, _CXC, _CYC = _col_patterns()
    x1 = boxes_ref[:, 0:1]
    y1 = boxes_ref[:, 1:2]
    x2 = boxes_ref[:, 2:3]
    y2 = boxes_ref[:, 3:4]
    area = (y2 - y1) * (x2 - x1)
    lvl = ((area >= 12544.0).astype(jnp.int32)
           + (area >= 50176.0).astype(jnp.int32)
           + (area >= 200704.0).astype(jnp.int32))
    scale = jnp.where(lvl == 0, 0.25,
                      jnp.where(lvl == 1, 0.125,
                                jnp.where(lvl == 2, 0.0625, 0.03125)))
    hw = jnp.right_shift(jnp.int32(256), lvl)
    hwf = hw.astype(jnp.float32)
    base = (jnp.where(lvl >= 1, 131072, 0)
            + jnp.where(lvl >= 2, 32768, 0)
            + jnp.where(lvl >= 3, 8192, 0))
    b = jnp.where(pid >= 4, 1, 0)

    def axis(lo, hi, ct, corner):
        los = lo * scale
        his = hi * scale
        ext = jnp.maximum(his - los, 1.0)
        bw = ext / float(_POOL)
        t = los + jnp.asarray(ct) * bw
        validf = ((t > -1.0) & (t < hwf)).astype(jnp.float32)
        tc = jnp.clip(t, 0.0, hwf - 1.0)
        i0 = jnp.floor(tc).astype(jnp.int32)
        i1 = jnp.minimum(i0 + 1, hw - 1)
        frac = tc - i0.astype(jnp.float32)
        cmask = jnp.asarray(corner) == 1
        sel = jnp.where(cmask, i1, i0)
        wgt = jnp.where(cmask, frac, 1.0 - frac) * validf
        return sel, wgt

    xsel, wx = axis(x1, x2, _CXT, _CXC)
    ysel, wy = axis(y1, y2, _CYT, _CYC)
    idx_ref[...] = base + (b * hw + ysel) * hw + xsel
    w_ref[...] = wx * wy * 0.25


def _phase1(boxes):
    return pl.pallas_call(
        _p1_body,
        grid=(8,),
        in_specs=[pl.BlockSpec((128, 4), lambda i: (i, 0))],
        out_specs=[pl.BlockSpec((128, _NP), lambda i: (i, 0)),
                   pl.BlockSpec((128, _NP), lambda i: (i, 0))],
        out_shape=[jax.ShapeDtypeStruct((_R, _NP), jnp.int32),
                   jax.ShapeDtypeStruct((_R, _NP), jnp.float32)],
    )(boxes)


def _lane_bcast(v, j):
    idxs = jnp.full((16, 1), j, jnp.int32)
    return lax.gather(
        v, idxs,
        lax.GatherDimensionNumbers(offset_dims=(), collapsed_slice_dims=(0,),
                                   start_index_map=(0,)),
        (1,), mode=lax.GatherScatterMode.PROMISE_IN_BOUNDS)


_NCH = _RPW * _NCHUNK
_IDXN = _RPW * _NP


def _sc_body(idx_hbm, w_hbm, table_hbm, out_hbm,
             idx_v, w_v, rows0, rows1, stage0, stage1,
             gsem0, gsem1, osem0, osem1):
    nc = 2
    wid = lax.axis_index("s") * nc + lax.axis_index("c")
    rows = (rows0, rows1)
    stages = (stage0, stage1)
    gsems = (gsem0, gsem1)
    osems = (osem0, osem1)

    pltpu.sync_copy(idx_hbm.at[pl.ds(wid * _IDXN, _IDXN)], idx_v)
    pltpu.sync_copy(w_hbm.at[pl.ds(wid * _IDXN, _IDXN)], w_v)

    def start_gather(c, buf):
        off = (c // _NCHUNK) * _NP + (c % _NCHUNK) * _CHUNK
        pltpu.async_copy(table_hbm.at[idx_v.at[pl.ds(off, _CHUNK)]],
                         rows[buf], gsems[buf])

    def wait_gather(buf):
        pltpu.make_async_copy(table_hbm.at[pl.ds(0, _CHUNK)], rows[buf],
                              gsems[buf]).wait()

    def wait_out(buf):
        pltpu.make_async_copy(out_hbm.at[pl.ds(0, 8)], stages[buf],
                              osems[buf]).wait()

    def compute_chunk(c, buf, first):
        rl = c // _NCHUNK
        cc = c % _NCHUNK
        wbase = rl * _NP + cc * _CHUNK
        wait_gather(buf)

        @pl.when(jnp.logical_not(first))
        def _():
            wait_out(buf)

        def bin_body(bl, _):
            wvec = w_v[pl.ds(wbase + bl * 16, 16)]
            acc = [jnp.zeros((16,), jnp.float32) for _ in range(16)]
            for j in range(16):
                wj = _lane_bcast(wvec, j)
                for k in range(16):
                    acc[k] = acc[k] + wj * rows[buf][bl * 16 + j, pl.ds(k * 16, 16)]
            for k in range(16):
                stages[buf][bl, pl.ds(k * 16, 16)] = acc[k]
            return ()

        lax.fori_loop(0, 7, bin_body, (), unroll=False)
        r = wid * _RPW + rl
        pltpu.async_copy(stages[buf],
                         out_hbm.at[pl.ds(r * 56 + cc * 8, 8)], osems[buf])

    start_gather(jnp.int32(0), 0)

    def step(g, _):
        c0 = g * 2
        start_gather(c0 + 1, 1)
        compute_chunk(c0, 0, g == 0)

        @pl.when(c0 + 2 < _NCH)
        def _():
            start_gather(c0 + 2, 0)

        compute_chunk(c0 + 1, 1, g == 0)
        return ()

    lax.fori_loop(0, _NCH // 2, step, (), unroll=False)
    wait_out(0)
    wait_out(1)


def _phase2(idx, w, table):
    mesh = plsc.VectorSubcoreMesh(core_axis_name="c", subcore_axis_name="s")
    f = functools.partial(
        pl.kernel, _sc_body, mesh=mesh,
        out_type=jax.ShapeDtypeStruct((_R * 56, _C), jnp.float32),
        scratch_types=[
            pltpu.VMEM((_IDXN,), jnp.int32),
            pltpu.VMEM((_IDXN,), jnp.float32),
            pltpu.VMEM((_CHUNK, _C), jnp.float32),
            pltpu.VMEM((_CHUNK, _C), jnp.float32),
            pltpu.VMEM((8, _C), jnp.float32),
            pltpu.VMEM((8, _C), jnp.float32),
            pltpu.SemaphoreType.DMA,
            pltpu.SemaphoreType.DMA,
            pltpu.SemaphoreType.DMA,
            pltpu.SemaphoreType.DMA,
        ])()
    return f(idx, w, table)


def kernel(feat_p2, feat_p3, feat_p4, feat_p5, boxes0, boxes1):
    feats = [feat_p2, feat_p3, feat_p4, feat_p5]
    table = jnp.concatenate(
        [jnp.transpose(f, (0, 2, 3, 1)).reshape(-1, _C) for f in feats], axis=0)
    boxes = jnp.concatenate([boxes0, boxes1], axis=0)
    idx, w = _phase1(boxes)
    out_rows = _phase2(idx.reshape(-1), w.reshape(-1), table)
    out_rows = out_rows.reshape(_R, _POOL, 8, _C)[:, :, :_POOL, :]
    return jnp.transpose(out_rows, (0, 3, 1, 2))
